# baseline (device time: 24965 ns/iter reference)
import jax
import jax.numpy as jnp
from jax import lax
from jax.experimental import pallas as pl
from jax.experimental.pallas import tpu as pltpu

N_DEV = 4
B, SQ, SKV, DH = 2, 128, 128, 64
H_LOC = 4
D_MODEL = 512
SCALE = 0.125


def kernel(x, Wq, K_ext, V_ext, Wo):
    my_pos = lax.axis_index("i")
    k_loc = lax.dynamic_slice(
        K_ext, (0, 0, my_pos * H_LOC, 0), (B, SKV, H_LOC, DH))
    v_loc = lax.dynamic_slice(
        V_ext, (0, 0, my_pos * H_LOC, 0), (B, SKV, H_LOC, DH))

    def body(x_ref, wq_ref, k_ref, v_ref, wo_ref, out_ref,
             comm_ref, send_sems, recv_sems):
        my = lax.axis_index("i")
        p1 = jnp.bitwise_xor(my, 1)
        p2 = 3 - my

        barrier_sem = pltpu.get_barrier_semaphore()
        for nbr in (p1, p2):
            pl.semaphore_signal(barrier_sem, inc=1, device_id=(nbr,),
                                device_id_type=pl.DeviceIdType.MESH)
        pl.semaphore_wait(barrier_sem, 2)

        bf16 = jnp.bfloat16
        wq = wq_ref[...].astype(bf16)
        wo = wo_ref[...].astype(bf16)
        for b in range(B):
            xb = x_ref[b].astype(bf16)
            q = jnp.dot(xb, wq,
                        preferred_element_type=jnp.float32).astype(bf16)
            kb = k_ref[b].astype(bf16)
            vb = v_ref[b].astype(bf16)
            ctx = []
            for h in range(H_LOC):
                qh = q[:, h * DH:(h + 1) * DH]
                kh = kb[:, h, :]
                vh = vb[:, h, :]
                s = lax.dot_general(
                    qh, kh, (((1,), (1,)), ((), ())),
                    preferred_element_type=jnp.float32) * SCALE
                s = s - jnp.max(s, axis=-1, keepdims=True)
                w = jnp.exp(s)
                w = w / jnp.sum(w, axis=-1, keepdims=True)
                ctx.append(jnp.dot(w.astype(bf16), vh,
                                   preferred_element_type=jnp.float32
                                   ).astype(bf16))
            ctx_b = jnp.concatenate(ctx, axis=1)
            out_ref[b, :, :] = jnp.dot(ctx_b, wo,
                                       preferred_element_type=jnp.float32)

        for step, partner in enumerate((p1, p2)):
            rdma = pltpu.make_async_remote_copy(
                src_ref=out_ref,
                dst_ref=comm_ref.at[step],
                send_sem=send_sems.at[step],
                recv_sem=recv_sems.at[step],
                device_id=(partner,),
                device_id_type=pl.DeviceIdType.MESH,
            )
            rdma.start()
            rdma.wait()
            out_ref[...] = out_ref[...] + comm_ref[step]

        def _exit(sem):
            for nbr in (p1, p2):
                pl.semaphore_signal(sem, inc=1, device_id=(nbr,),
                                    device_id_type=pl.DeviceIdType.MESH)
            pl.semaphore_wait(sem, 2)
        pl.run_scoped(_exit, pltpu.SemaphoreType.REGULAR)

    return pl.pallas_call(
        body,
        out_shape=jax.ShapeDtypeStruct((B, SQ, D_MODEL), jnp.float32),
        in_specs=[pl.BlockSpec(memory_space=pltpu.VMEM)] * 5,
        out_specs=pl.BlockSpec(memory_space=pltpu.VMEM),
        scratch_shapes=[
            pltpu.VMEM((2, B, SQ, D_MODEL), jnp.float32),
            pltpu.SemaphoreType.DMA((2,)),
            pltpu.SemaphoreType.DMA((2,)),
        ],
        compiler_params=pltpu.CompilerParams(collective_id=0),
    )(x, Wq, k_loc, v_loc, Wo)


# device time: 19377 ns/iter; 1.2884x vs baseline; 1.2884x over previous
import jax
import jax.numpy as jnp
from jax import lax
from jax.experimental import pallas as pl
from jax.experimental.pallas import tpu as pltpu

N_DEV = 4
B, SQ, SKV, DH = 2, 128, 128, 64
H_LOC = 4
D_MODEL = 512
SCALE = 0.125


def kernel(x, Wq, K_ext, V_ext, Wo):
    my_pos = lax.axis_index("i")
    k_loc = lax.dynamic_slice(
        K_ext, (0, 0, my_pos * H_LOC, 0), (B, SKV, H_LOC, DH))
    v_loc = lax.dynamic_slice(
        V_ext, (0, 0, my_pos * H_LOC, 0), (B, SKV, H_LOC, DH))

    def body(x_ref, wq_ref, k_ref, v_ref, wo_ref, out_ref,
             sbuf, rbuf, send_sems, recv_sems):
        my = lax.axis_index("i")
        p1 = jnp.bitwise_xor(my, 1)
        p2 = 3 - my

        barrier_sem = pltpu.get_barrier_semaphore()
        for nbr in (p1, p2):
            pl.semaphore_signal(barrier_sem, inc=1, device_id=(nbr,),
                                device_id_type=pl.DeviceIdType.MESH)
        pl.semaphore_wait(barrier_sem, 2)

        bf16 = jnp.bfloat16
        wq = wq_ref[...].astype(bf16)
        wo = wo_ref[...].astype(bf16)
        for b in range(B):
            xb = x_ref[b].astype(bf16)
            q = jnp.dot(xb, wq,
                        preferred_element_type=jnp.float32).astype(bf16)
            kb = k_ref[b].astype(bf16)
            vb = v_ref[b].astype(bf16)
            ctx = []
            for h in range(H_LOC):
                qh = q[:, h * DH:(h + 1) * DH]
                kh = kb[:, h, :]
                vh = vb[:, h, :]
                s = lax.dot_general(
                    qh, kh, (((1,), (1,)), ((), ())),
                    preferred_element_type=jnp.float32) * SCALE
                s = s - jnp.max(s, axis=-1, keepdims=True)
                w = jnp.exp(s)
                w = w / jnp.sum(w, axis=-1, keepdims=True)
                ctx.append(jnp.dot(w.astype(bf16), vh,
                                   preferred_element_type=jnp.float32
                                   ).astype(bf16))
            ctx_b = jnp.concatenate(ctx, axis=1)
            partial_b = jnp.dot(ctx_b, wo, preferred_element_type=jnp.float32)
            out_ref[b, :, :] = partial_b
            sbuf[0, b, :, :] = partial_b.astype(bf16)

        rdmas = []
        for step, partner in enumerate((p1, p2)):
            rdma = pltpu.make_async_remote_copy(
                src_ref=sbuf.at[step],
                dst_ref=rbuf.at[step],
                send_sem=send_sems.at[step],
                recv_sem=recv_sems.at[step],
                device_id=(partner,),
                device_id_type=pl.DeviceIdType.MESH,
            )
            rdmas.append(rdma)
            rdma.start()
            rdma.wait_recv()
            summed = out_ref[...] + rbuf[step].astype(jnp.float32)
            out_ref[...] = summed
            if step == 0:
                sbuf[1, :, :, :] = summed.astype(bf16)
        for rdma in rdmas:
            rdma.wait_send()

        def _exit(sem):
            for nbr in (p1, p2):
                pl.semaphore_signal(sem, inc=1, device_id=(nbr,),
                                    device_id_type=pl.DeviceIdType.MESH)
            pl.semaphore_wait(sem, 2)
        pl.run_scoped(_exit, pltpu.SemaphoreType.REGULAR)

    return pl.pallas_call(
        body,
        out_shape=jax.ShapeDtypeStruct((B, SQ, D_MODEL), jnp.float32),
        in_specs=[pl.BlockSpec(memory_space=pltpu.VMEM)] * 5,
        out_specs=pl.BlockSpec(memory_space=pltpu.VMEM),
        scratch_shapes=[
            pltpu.VMEM((2, B, SQ, D_MODEL), jnp.bfloat16),
            pltpu.VMEM((2, B, SQ, D_MODEL), jnp.bfloat16),
            pltpu.SemaphoreType.DMA((2,)),
            pltpu.SemaphoreType.DMA((2,)),
        ],
        compiler_params=pltpu.CompilerParams(collective_id=0),
    )(x, Wq, k_loc, v_loc, Wo)


# device time: 13220 ns/iter; 1.8884x vs baseline; 1.4657x over previous
import jax
import jax.numpy as jnp
from jax import lax
from jax.experimental import pallas as pl
from jax.experimental.pallas import tpu as pltpu

N_DEV = 4
B, SQ, SKV, DH = 2, 128, 128, 64
H_LOC = 4
HD_LOC = H_LOC * DH
D_MODEL = 512
SCALE = 0.125


def kernel(x, Wq, K_ext, V_ext, Wo):
    my_pos = lax.axis_index("i")
    def _slab(a):
        s = lax.dynamic_slice(a, (0, 0, my_pos * H_LOC, 0),
                              (B, SKV, H_LOC, DH))
        return s.reshape(B, SKV, HD_LOC).astype(jnp.bfloat16)

    k_loc = _slab(K_ext)
    v_loc = _slab(V_ext)

    def body(x_ref, wq_ref, k_ref, v_ref, wo_ref, out_ref,
             sbuf, rbuf, send_sems, recv_sems):
        my = lax.axis_index("i")
        p1 = jnp.bitwise_xor(my, 1)
        p2 = 3 - my

        barrier_sem = pltpu.get_barrier_semaphore()
        for nbr in (p1, p2):
            pl.semaphore_signal(barrier_sem, inc=1, device_id=(nbr,),
                                device_id_type=pl.DeviceIdType.MESH)

        def make_rdma(step, b, half, partner):
            slot = (step * B + b) * 2 + half
            return pltpu.make_async_remote_copy(
                src_ref=sbuf.at[slot],
                dst_ref=rbuf.at[slot],
                send_sem=send_sems.at[slot],
                recv_sem=recv_sems.at[slot],
                device_id=(partner,),
                device_id_type=pl.DeviceIdType.MESH,
            )

        HALF = D_MODEL // 2
        cols = (slice(0, HALF), slice(HALF, D_MODEL))

        bf16 = jnp.bfloat16
        wq = wq_ref[...].astype(bf16)
        wo = wo_ref[...].astype(bf16)
        x1 = []
        for b in range(B):
            xb = x_ref[b].astype(bf16)
            q = jnp.dot(xb, wq,
                        preferred_element_type=jnp.float32).astype(bf16)
            kb = k_ref[b]
            vb = v_ref[b]
            ctx = []
            for h in range(H_LOC):
                sl = slice(h * DH, (h + 1) * DH)
                s = lax.dot_general(
                    q[:, sl], kb[:, sl], (((1,), (1,)), ((), ())),
                    preferred_element_type=jnp.float32) * SCALE
                w = jnp.exp(s)
                denom = jnp.sum(w, axis=-1, keepdims=True)
                ctx_h = jnp.dot(w.astype(bf16), vb[:, sl],
                                preferred_element_type=jnp.float32)
                ctx.append((ctx_h / denom).astype(bf16))
            ctx_b = jnp.concatenate(ctx, axis=1)
            partial = jnp.dot(ctx_b, wo, preferred_element_type=jnp.float32)
            sbuf[2 * b, :, :] = partial[:, cols[0]].astype(bf16)
            sbuf[2 * b + 1, :, :] = partial[:, cols[1]].astype(bf16)
            if b == 0:
                pl.semaphore_wait(barrier_sem, 2)
            for half, partner in ((0, p1), (1, p2)):
                rdma = make_rdma(0, b, half, partner)
                x1.append(rdma)
                rdma.start()

        x2 = []
        for b in range(B):
            for half in (0, 1):
                x1[2 * b + half].wait_recv()
                sbuf[(B + b) * 2 + half, :, :] = (
                    sbuf[2 * b + half, :, :] + rbuf[2 * b + half, :, :])
                rdma = make_rdma(1, b, half, p2 if half == 0 else p1)
                x2.append(rdma)
                rdma.start()

        for b in range(B):
            for half in (0, 1):
                x2[2 * b + half].wait_recv()
                out_ref[b, :, cols[half]] = (
                    sbuf[(B + b) * 2 + half, :, :]
                    + rbuf[(B + b) * 2 + half, :, :]).astype(jnp.float32)

        for rdma in x1 + x2:
            rdma.wait_send()

        def _exit(sem):
            for nbr in (p1, p2):
                pl.semaphore_signal(sem, inc=1, device_id=(nbr,),
                                    device_id_type=pl.DeviceIdType.MESH)
            pl.semaphore_wait(sem, 2)
        pl.run_scoped(_exit, pltpu.SemaphoreType.REGULAR)

    return pl.pallas_call(
        body,
        out_shape=jax.ShapeDtypeStruct((B, SQ, D_MODEL), jnp.float32),
        in_specs=[pl.BlockSpec(memory_space=pltpu.VMEM)] * 5,
        out_specs=pl.BlockSpec(memory_space=pltpu.VMEM),
        scratch_shapes=[
            pltpu.VMEM((4 * B, SQ, D_MODEL // 2), jnp.bfloat16),
            pltpu.VMEM((4 * B, SQ, D_MODEL // 2), jnp.bfloat16),
            pltpu.SemaphoreType.DMA((4 * B,)),
            pltpu.SemaphoreType.DMA((4 * B,)),
        ],
        compiler_params=pltpu.CompilerParams(collective_id=0),
    )(x, Wq, k_loc, v_loc, Wo)
